# baseline (device time: 14994 ns/iter reference)
import jax
import jax.numpy as jnp
from jax import lax
from jax.experimental import pallas as pl
from jax.experimental.pallas import tpu as pltpu

BS = 64
DY = 640
M = 1024
KY = DY // BS
KF = (M - DY) // BS


def kernel(x):
    m, n = x.shape
    assert m == M
    out_dtype = jnp.bfloat16

    def body(x_ref, out_ref, se, re_, sx, rx):
        my_x = lax.axis_index("x")
        my_y = lax.axis_index("y")
        my_z = lax.axis_index("z")
        ynbr = (my_x, 1 - my_y, my_z)
        xnbr = (1 - my_x, my_y, my_z)

        barrier_sem = pltpu.get_barrier_semaphore()
        for nbr in (ynbr, xnbr):
            pl.semaphore_signal(
                barrier_sem, inc=1, device_id=nbr,
                device_id_type=pl.DeviceIdType.MESH,
            )
        pl.semaphore_wait(barrier_sem, 2)

        own = my_y * m
        rem = (1 - my_y) * m

        def roff(k):
            return (1 - my_x) * (k * BS) + my_x * (m - (k + 1) * BS)

        def yrdma(k):
            r = roff(k)
            return pltpu.make_async_remote_copy(
                src_ref=out_ref.at[pl.ds(own + r, BS), :],
                dst_ref=out_ref.at[pl.ds(own + r, BS), :],
                send_sem=se.at[k],
                recv_sem=re_.at[k],
                device_id=ynbr,
                device_id_type=pl.DeviceIdType.MESH,
            )

        def xrdma(k):
            r = roff(k)
            return pltpu.make_async_remote_copy(
                src_ref=out_ref.at[pl.ds(rem + r, BS), :],
                dst_ref=out_ref.at[pl.ds(rem + r, BS), :],
                send_sem=sx.at[k],
                recv_sem=rx.at[k],
                device_id=xnbr,
                device_id_type=pl.DeviceIdType.MESH,
            )

        for k in range(KY):
            r = roff(k)
            out_ref[pl.ds(own + r, BS), :] = x_ref[pl.ds(r, BS), :].astype(
                out_dtype
            )
            yrdma(k).start()

        tail = (1 - my_x) * DY
        out_ref[pl.ds(own + tail, m - DY), :] = x_ref[
            pl.ds(tail, m - DY), :
        ].astype(out_dtype)

        for k in range(KF):
            yrdma(k).wait_recv()
            xrdma(k).start()

        for k in range(KF, KY):
            yrdma(k).wait_recv()
        for k in range(KF):
            xrdma(k).wait_recv()
        for k in range(KY):
            yrdma(k).wait_send()
        for k in range(KF):
            xrdma(k).wait_send()

    return pl.pallas_call(
        body,
        out_shape=jax.ShapeDtypeStruct((2 * m, n), out_dtype),
        in_specs=[pl.BlockSpec(memory_space=pltpu.VMEM)],
        out_specs=pl.BlockSpec(memory_space=pltpu.VMEM),
        scratch_shapes=[
            pltpu.SemaphoreType.DMA((KY,)),
            pltpu.SemaphoreType.DMA((KY,)),
            pltpu.SemaphoreType.DMA((KF,)),
            pltpu.SemaphoreType.DMA((KF,)),
        ],
        compiler_params=pltpu.CompilerParams(collective_id=0),
    )(x)
